# baseline (speedup 1.0000x reference)
import functools

import jax
import jax.numpy as jnp
from jax import lax
from jax.experimental import pallas as pl
from jax.experimental.pallas import tpu as pltpu
from jax.experimental.pallas import tpu_sc as plsc

E = 8
G = 4
S = 2048
D = 1024
DFF = 768
TILE = 256
C = S * 2 + G * TILE
NT = C // TILE
NW = 32
TPW = S // NW


def _router_kernel(mg_ref, x_ref, gw_ref, pos_ref, w_ref, tmeta_ref):
    x = x_ref[...]
    logits = lax.dot_general(x, gw_ref[...], (((1,), (1,)), ((), ())),
                             preferred_element_type=jnp.float32)
    m = jnp.max(logits, axis=1, keepdims=True)
    ex = jnp.exp(logits - m)
    probs = ex / jnp.sum(ex, axis=1, keepdims=True)

    i1 = jnp.argmax(probs, axis=1)
    v1 = jnp.max(probs, axis=1)
    iota_e = lax.broadcasted_iota(jnp.int32, probs.shape, 1)
    masked = jnp.where(iota_e == i1[:, None], -jnp.inf, probs)
    i2 = jnp.argmax(masked, axis=1)
    v2 = jnp.max(masked, axis=1)
    denom = v1 + v2
    w0 = v1 / denom
    w1 = v2 / denom

    e1 = (iota_e == i1[:, None]).astype(jnp.float32)
    e2 = (iota_e == i2[:, None]).astype(jnp.float32)
    iota_eg_e = lax.broadcasted_iota(jnp.int32, (E, G), 0)
    iota_eg_g = lax.broadcasted_iota(jnp.int32, (E, G), 1)
    mgv = jnp.zeros((E, G), jnp.int32)
    for e in range(E):
        mgv = mgv + jnp.where(iota_eg_e == e, mg_ref[e], 0)
    gm = (iota_eg_g == mgv).astype(jnp.float32)
    m1 = lax.dot_general(e1, gm, (((1,), (0,)), ((), ())),
                         preferred_element_type=jnp.float32)
    m2 = lax.dot_general(e2, gm, (((1,), (0,)), ((), ())),
                         preferred_element_type=jnp.float32)

    ia = lax.broadcasted_iota(jnp.int32, (S, S), 0)
    ib = lax.broadcasted_iota(jnp.int32, (S, S), 1)
    ltri = (ib < ia).astype(jnp.float32)
    r0 = lax.dot_general(ltri, m1, (((1,), (0,)), ((), ())),
                         preferred_element_type=jnp.float32)
    r1 = lax.dot_general(ltri, m2, (((1,), (0,)), ((), ())),
                         preferred_element_type=jnp.float32)

    c0 = jnp.sum(m1, axis=0, keepdims=True)
    c1 = jnp.sum(m2, axis=0, keepdims=True)
    ctot = c0 + c1
    pc = jnp.ceil(ctot / TILE) * TILE
    ig_a = lax.broadcasted_iota(jnp.int32, (G, G), 0)
    ig_b = lax.broadcasted_iota(jnp.int32, (G, G), 1)
    ltri_g = (ig_a < ig_b).astype(jnp.float32)
    st = lax.dot_general(pc, ltri_g, (((1,), (0,)), ((), ())),
                         preferred_element_type=jnp.float32)

    pos0 = jnp.sum(m1 * (st + r0), axis=1)
    pos1 = jnp.sum(m2 * (st + c0 + r1), axis=1)

    iota_se = lax.broadcasted_iota(jnp.int32, (S, E), 1)
    pos_mat = (jnp.where(iota_se == 0, pos0[:, None], 0.0)
               + jnp.where(iota_se == 1, pos1[:, None], 0.0))
    w_mat = (jnp.where(iota_se == 0, w0[:, None], 0.0)
             + jnp.where(iota_se == 1, w1[:, None], 0.0))
    pos_ref[...] = pos_mat.astype(jnp.int32)
    w_ref[...] = w_mat

    ti = (lax.broadcasted_iota(jnp.int32, (8, 128), 1) * TILE).astype(jnp.float32)
    tg = jnp.zeros((8, 128), jnp.int32)
    for g in range(1, G):
        tg = tg + jnp.where(ti >= st[0:1, g:g + 1], 1, 0)
    nact = (jnp.sum(pc) / TILE).astype(jnp.int32)
    tmeta_ref[0:1, :] = tg[0:1, :]
    tmeta_ref[1:2, :] = jnp.full((1, 128), nact, jnp.int32)


def _router(x, gate_weight, merge_groups):
    grid_spec = pltpu.PrefetchScalarGridSpec(
        num_scalar_prefetch=1,
        grid=(1,),
        in_specs=[
            pl.BlockSpec((S, D), lambda i, mg: (0, 0)),
            pl.BlockSpec((E, D), lambda i, mg: (0, 0)),
        ],
        out_specs=[
            pl.BlockSpec((S, E), lambda i, mg: (0, 0)),
            pl.BlockSpec((S, E), lambda i, mg: (0, 0)),
            pl.BlockSpec((8, 128), lambda i, mg: (0, 0)),
        ],
    )
    return pl.pallas_call(
        _router_kernel,
        grid_spec=grid_spec,
        out_shape=[
            jax.ShapeDtypeStruct((S, E), jnp.int32),
            jax.ShapeDtypeStruct((S, E), jnp.float32),
            jax.ShapeDtypeStruct((8, 128), jnp.int32),
        ],
    )(merge_groups, x, gate_weight)


def _make_scatter():
    mesh = plsc.VectorSubcoreMesh(core_axis_name="c", subcore_axis_name="s")

    @functools.partial(
        pl.kernel, mesh=mesh,
        out_type=[
            jax.ShapeDtypeStruct((C, D), jnp.float32),
            jax.ShapeDtypeStruct((C,), jnp.float32),
        ],
        scratch_types=[
            pltpu.VMEM((TPW,), jnp.int32),
            pltpu.VMEM((TPW,), jnp.int32),
            pltpu.VMEM((TPW,), jnp.float32),
            pltpu.VMEM((TPW,), jnp.float32),
            pltpu.VMEM((TPW, D), jnp.float32),
            pltpu.SemaphoreType.DMA,
        ],
    )
    def scatter(x_hbm, post_hbm, wt_hbm, xs_hbm, ws_hbm,
                idx0_v, idx1_v, w0_v, w1_v, rows_v, sem):
        wid = lax.axis_index("s") * 2 + lax.axis_index("c")
        base = wid * TPW
        pltpu.sync_copy(post_hbm.at[0, pl.ds(base, TPW)], idx0_v)
        pltpu.sync_copy(post_hbm.at[1, pl.ds(base, TPW)], idx1_v)
        pltpu.sync_copy(wt_hbm.at[0, pl.ds(base, TPW)], w0_v)
        pltpu.sync_copy(wt_hbm.at[1, pl.ds(base, TPW)], w1_v)
        pltpu.sync_copy(x_hbm.at[pl.ds(base, TPW)], rows_v)
        pltpu.async_copy(rows_v, xs_hbm.at[idx0_v], sem).wait()
        pltpu.async_copy(rows_v, xs_hbm.at[idx1_v], sem).wait()
        pltpu.async_copy(w0_v, ws_hbm.at[idx0_v], sem).wait()
        pltpu.async_copy(w1_v, ws_hbm.at[idx1_v], sem).wait()

    return scatter


def _ffn_kernel(tmeta_ref, dom_ref, xs_ref, ws_ref, gu_ref, dn_ref, out_ref):
    i = pl.program_id(0)
    nact = tmeta_ref[1, 0]

    @pl.when(i < nact)
    def _body():
        xt = xs_ref[...]
        gu = lax.dot_general(xt, gu_ref[0], (((1,), (1,)), ((), ())),
                             preferred_element_type=jnp.float32)
        gate_h = gu[:, :DFF]
        up_h = gu[:, DFF:]
        h = gate_h * lax.logistic(gate_h) * up_h
        h = h * ws_ref[...]
        out_ref[...] = lax.dot_general(h, dn_ref[0], (((1,), (1,)), ((), ())),
                                       preferred_element_type=jnp.float32)


def _ffn(xs, ws2d, gate_up_proj, down_proj, tmeta, dominant_experts):
    grid_spec = pltpu.PrefetchScalarGridSpec(
        num_scalar_prefetch=2,
        grid=(NT,),
        in_specs=[
            pl.BlockSpec((TILE, D), lambda i, tm, dom: (i, 0)),
            pl.BlockSpec((TILE, 1), lambda i, tm, dom: (i, 0)),
            pl.BlockSpec((1, 2 * DFF, D), lambda i, tm, dom: (dom[tm[0, i]], 0, 0)),
            pl.BlockSpec((1, D, DFF), lambda i, tm, dom: (dom[tm[0, i]], 0, 0)),
        ],
        out_specs=pl.BlockSpec((TILE, D), lambda i, tm, dom: (i, 0)),
    )
    return pl.pallas_call(
        _ffn_kernel,
        grid_spec=grid_spec,
        out_shape=jax.ShapeDtypeStruct((C, D), jnp.float32),
        compiler_params=pltpu.CompilerParams(
            dimension_semantics=("arbitrary",),
        ),
    )(tmeta, dominant_experts, xs, ws2d, gate_up_proj, down_proj)


CHUNK = 32


def _make_combine():
    mesh = plsc.VectorSubcoreMesh(core_axis_name="c", subcore_axis_name="s")

    @functools.partial(
        pl.kernel, mesh=mesh,
        out_type=jax.ShapeDtypeStruct((S, D), jnp.float32),
        scratch_types=[
            pltpu.VMEM((CHUNK,), jnp.int32),
            pltpu.VMEM((CHUNK,), jnp.int32),
            pltpu.VMEM((CHUNK, D), jnp.float32),
            pltpu.VMEM((CHUNK, D), jnp.float32),
            pltpu.SemaphoreType.DMA,
        ],
    )
    def combine(ffn_hbm, post_hbm, out_hbm,
                idx0_v, idx1_v, r0_v, r1_v, sem):
        wid = lax.axis_index("s") * 2 + lax.axis_index("c")

        for chunk in range(TPW // CHUNK):
            base = wid * TPW + chunk * CHUNK
            pltpu.sync_copy(post_hbm.at[0, pl.ds(base, CHUNK)], idx0_v)
            pltpu.sync_copy(post_hbm.at[1, pl.ds(base, CHUNK)], idx1_v)
            pltpu.async_copy(ffn_hbm.at[idx0_v], r0_v, sem).wait()
            pltpu.async_copy(ffn_hbm.at[idx1_v], r1_v, sem).wait()

            def row_body(i, _):
                def col_body(l, _):
                    a = r0_v[i, pl.ds(l * 16, 16)]
                    b = r1_v[i, pl.ds(l * 16, 16)]
                    r0_v[i, pl.ds(l * 16, 16)] = a + b
                    return 0

                lax.fori_loop(0, D // 16, col_body, 0, unroll=False)
                return 0

            lax.fori_loop(0, CHUNK, row_body, 0, unroll=False)
            pltpu.sync_copy(r0_v, out_hbm.at[pl.ds(base, CHUNK)])

    return combine


def kernel(hidden_states, gate_weight, gate_up_proj, down_proj, merge_groups, dominant_experts):
    b, s, d = hidden_states.shape
    x = hidden_states.reshape(s, d)

    pos, w, tmeta = _router(x, gate_weight, merge_groups)
    pos_t = pos.T
    w_t = w.T

    xs, ws = _make_scatter()(x, pos_t, w_t)
    ffn = _ffn(xs, ws.reshape(C, 1), gate_up_proj, down_proj, tmeta, dominant_experts)
    out = _make_combine()(ffn, pos_t)
    return out.reshape(b, s, d)


# speedup vs baseline: 2.4827x; 2.4827x over previous
import functools

import jax
import jax.numpy as jnp
from jax import lax
from jax.experimental import pallas as pl
from jax.experimental.pallas import tpu as pltpu

E = 8
TOP_K = 2
TM = 256


def _moe_kernel(mg_ref, dom_ref, x_ref, gw_ref, *rest, num_groups):
    gu_refs = rest[:num_groups]
    dn_refs = rest[num_groups:2 * num_groups]
    out_ref = rest[2 * num_groups]

    xt = x_ref[...]

    logits = lax.dot_general(
        xt, gw_ref[...], (((1,), (1,)), ((), ())),
        preferred_element_type=jnp.float32)
    m = jnp.max(logits, axis=1, keepdims=True)
    ex = jnp.exp(logits - m)
    probs = ex / jnp.sum(ex, axis=1, keepdims=True)

    i1 = jnp.argmax(probs, axis=1)
    v1 = jnp.max(probs, axis=1)
    iota = lax.broadcasted_iota(jnp.int32, probs.shape, 1)
    masked = jnp.where(iota == i1[:, None], -jnp.inf, probs)
    i2 = jnp.argmax(masked, axis=1)
    v2 = jnp.max(masked, axis=1)
    denom = v1 + v2

    acc = None
    for g in range(num_groups):
        wg = jnp.zeros_like(v1)
        for e in range(E):
            in_g = mg_ref[e] == g
            sel = jnp.where(i1 == e, v1, 0.0) + jnp.where(i2 == e, v2, 0.0)
            wg = wg + jnp.where(in_g, sel, 0.0)
        wg = wg / denom

        gu = lax.dot_general(
            xt, gu_refs[g][0], (((1,), (1,)), ((), ())),
            preferred_element_type=jnp.float32)
        dff = gu.shape[1] // 2
        gate_h = gu[:, :dff]
        up_h = gu[:, dff:]
        h = gate_h * lax.logistic(gate_h) * up_h
        out = lax.dot_general(
            h, dn_refs[g][0], (((1,), (1,)), ((), ())),
            preferred_element_type=jnp.float32)
        term = out * wg[:, None]
        acc = term if acc is None else acc + term

    out_ref[...] = acc


def kernel(hidden_states, gate_weight, gate_up_proj, down_proj, merge_groups, dominant_experts):
    b, s, d = hidden_states.shape
    x = hidden_states.reshape(s, d)
    num_groups = dominant_experts.shape[0]
    two_dff = gate_up_proj.shape[1]
    dff = down_proj.shape[2]
    n_t = s // TM

    def gu_spec(g):
        return pl.BlockSpec((1, two_dff, d), lambda t, mg, dom: (dom[g], 0, 0))

    def dn_spec(g):
        return pl.BlockSpec((1, d, dff), lambda t, mg, dom: (dom[g], 0, 0))

    grid_spec = pltpu.PrefetchScalarGridSpec(
        num_scalar_prefetch=2,
        grid=(n_t,),
        in_specs=[
            pl.BlockSpec((TM, d), lambda t, mg, dom: (t, 0)),
            pl.BlockSpec((E, d), lambda t, mg, dom: (0, 0)),
        ] + [gu_spec(g) for g in range(num_groups)]
          + [dn_spec(g) for g in range(num_groups)],
        out_specs=pl.BlockSpec((TM, d), lambda t, mg, dom: (t, 0)),
    )

    out = pl.pallas_call(
        functools.partial(_moe_kernel, num_groups=num_groups),
        grid_spec=grid_spec,
        out_shape=jax.ShapeDtypeStruct((s, d), x.dtype),
        compiler_params=pltpu.CompilerParams(
            dimension_semantics=("arbitrary",),
        ),
    )(merge_groups, dominant_experts, x, gate_weight,
      *([gate_up_proj] * num_groups), *([down_proj] * num_groups))
    return out.reshape(b, s, d)


# speedup vs baseline: 2.5203x; 1.0151x over previous
import functools

import jax
import jax.numpy as jnp
from jax import lax
from jax.experimental import pallas as pl
from jax.experimental.pallas import tpu as pltpu

E = 8
TOP_K = 2
TM = 512


def _moe_kernel(mg_ref, dom_ref, x_ref, gw_ref, *rest, num_groups):
    gu_refs = rest[:num_groups]
    dn_refs = rest[num_groups:2 * num_groups]
    out_ref = rest[2 * num_groups]

    xt = x_ref[...]

    logits = lax.dot_general(
        xt, gw_ref[...], (((1,), (1,)), ((), ())),
        preferred_element_type=jnp.float32)
    m = jnp.max(logits, axis=1, keepdims=True)
    ex = jnp.exp(logits - m)
    probs = ex / jnp.sum(ex, axis=1, keepdims=True)

    i1 = jnp.argmax(probs, axis=1)
    v1 = jnp.max(probs, axis=1)
    iota = lax.broadcasted_iota(jnp.int32, probs.shape, 1)
    masked = jnp.where(iota == i1[:, None], -jnp.inf, probs)
    i2 = jnp.argmax(masked, axis=1)
    v2 = jnp.max(masked, axis=1)
    denom = v1 + v2

    acc = None
    for g in range(num_groups):
        wg = jnp.zeros_like(v1)
        for e in range(E):
            in_g = mg_ref[e] == g
            sel = jnp.where(i1 == e, v1, 0.0) + jnp.where(i2 == e, v2, 0.0)
            wg = wg + jnp.where(in_g, sel, 0.0)
        wg = wg / denom

        gu = lax.dot_general(
            xt, gu_refs[g][0], (((1,), (1,)), ((), ())),
            preferred_element_type=jnp.float32)
        dff = gu.shape[1] // 2
        gate_h = gu[:, :dff]
        up_h = gu[:, dff:]
        h = gate_h * lax.logistic(gate_h) * up_h
        out = lax.dot_general(
            h, dn_refs[g][0], (((1,), (1,)), ((), ())),
            preferred_element_type=jnp.float32)
        term = out * wg[:, None]
        acc = term if acc is None else acc + term

    out_ref[...] = acc


def kernel(hidden_states, gate_weight, gate_up_proj, down_proj, merge_groups, dominant_experts):
    b, s, d = hidden_states.shape
    x = hidden_states.reshape(s, d)
    num_groups = dominant_experts.shape[0]
    two_dff = gate_up_proj.shape[1]
    dff = down_proj.shape[2]
    n_t = s // TM

    def gu_spec(g):
        return pl.BlockSpec((1, two_dff, d), lambda t, mg, dom: (dom[g], 0, 0))

    def dn_spec(g):
        return pl.BlockSpec((1, d, dff), lambda t, mg, dom: (dom[g], 0, 0))

    grid_spec = pltpu.PrefetchScalarGridSpec(
        num_scalar_prefetch=2,
        grid=(n_t,),
        in_specs=[
            pl.BlockSpec((TM, d), lambda t, mg, dom: (t, 0)),
            pl.BlockSpec((E, d), lambda t, mg, dom: (0, 0)),
        ] + [gu_spec(g) for g in range(num_groups)]
          + [dn_spec(g) for g in range(num_groups)],
        out_specs=pl.BlockSpec((TM, d), lambda t, mg, dom: (t, 0)),
    )

    out = pl.pallas_call(
        functools.partial(_moe_kernel, num_groups=num_groups),
        grid_spec=grid_spec,
        out_shape=jax.ShapeDtypeStruct((s, d), x.dtype),
        compiler_params=pltpu.CompilerParams(
            dimension_semantics=("arbitrary",),
        ),
    )(merge_groups, dominant_experts, x, gate_weight,
      *([gate_up_proj] * num_groups), *([down_proj] * num_groups))
    return out.reshape(b, s, d)
